# baseline (speedup 1.0000x reference)
import functools

import jax
import jax.numpy as jnp
from jax import lax
from jax.experimental import pallas as pl
from jax.experimental.pallas import tpu as pltpu
from jax.experimental.pallas import tpu_sc as plsc

DIM = 32
CHUNK = 128

_BITREV = [0, 8, 4, 12, 2, 10, 6, 14, 1, 9, 5, 13, 3, 11, 7, 15]


def kernel(u, i, user_emb, item_emb):
    B = u.shape[0]
    info = plsc.get_sparse_core_info()
    NC, NS = info.num_cores, info.num_subcores
    NW = NC * NS
    bpw = B // NW
    nchunk = bpw // CHUNK

    u3 = u.astype(jnp.int32).reshape(NW, nchunk, CHUNK)
    i3 = i.astype(jnp.int32).reshape(NW, nchunk, CHUNK)

    mesh = plsc.VectorSubcoreMesh(core_axis_name="c", subcore_axis_name="s")

    @functools.partial(
        pl.kernel,
        mesh=mesh,
        out_type=jax.ShapeDtypeStruct((B,), jnp.float32),
        compiler_params=pltpu.CompilerParams(use_tc_tiling_on_sc=False),
        scratch_types=[
            pltpu.VMEM((nchunk, CHUNK), jnp.int32),
            pltpu.VMEM((nchunk, CHUNK), jnp.int32),
            pltpu.VMEM((bpw, DIM), jnp.float32),
            pltpu.VMEM((bpw, DIM), jnp.float32),
            pltpu.VMEM((bpw,), jnp.float32),
            pltpu.SemaphoreType.DMA((2,)),
            pltpu.SemaphoreType.DMA((nchunk,)),
        ],
    )
    def sc_kernel(u_hbm, i_hbm, ue_hbm, ie_hbm, out_hbm,
                  uidx, iidx, urows, irows, outv, isem, gsem):
        wid = lax.axis_index("s") * NC + lax.axis_index("c")
        base = wid * bpw

        idx_cp = [
            pltpu.async_copy(u_hbm.at[wid], uidx, isem.at[0]),
            pltpu.async_copy(i_hbm.at[wid], iidx, isem.at[1]),
        ]
        idx_cp[0].wait()
        idx_cp[1].wait()

        copies = []
        for j in range(nchunk):
            sl = pl.ds(j * CHUNK, CHUNK)
            copies.append(pltpu.async_copy(
                ue_hbm.at[uidx.at[j]], urows.at[sl], gsem.at[j]))
            copies.append(pltpu.async_copy(
                ie_hbm.at[iidx.at[j]], irows.at[sl], gsem.at[j]))

        lane = lax.iota(jnp.int32, 16)
        perm = {k: lane ^ k for k in (8, 4, 2, 1)}
        msk = {k: (lane & k) == 0 for k in (8, 4, 2, 1)}

        def bfly(x, k):
            return x + jnp.take_along_axis(x, perm[k], axis=0)

        def block16(blk):
            v = []
            for r in _BITREV:
                b = blk * 16 + r
                a0 = urows[b, pl.ds(0, 16)]
                a1 = urows[b, pl.ds(16, 16)]
                c0 = irows[b, pl.ds(0, 16)]
                c1 = irows[b, pl.ds(16, 16)]
                v.append(bfly(a0 * c0 + a1 * c1, 8))
            for k in (4, 2, 1):
                v = [jnp.where(msk[2 * k], v[2 * t], v[2 * t + 1])
                     for t in range(len(v) // 2)]
                v = [bfly(x, k) for x in v]
            v = jnp.where(msk[1], v[0], v[1])
            outv[pl.ds(blk * 16, 16)] = v

        for j in range(nchunk):
            copies[2 * j].wait()
            copies[2 * j + 1].wait()

            def body(blk, carry):
                block16(j * (CHUNK // 16) + blk)
                return carry

            lax.fori_loop(0, CHUNK // 16, body, 0)

        pltpu.sync_copy(outv, out_hbm.at[pl.ds(base, bpw)])

    return sc_kernel(u3, i3, user_emb, item_emb)
